# Initial kernel scaffold; baseline (speedup 1.0000x reference)
import jax
import jax.numpy as jnp
from jax.experimental import pallas as pl


def kernel(input_image, U, V):
    raise NotImplementedError("write your pallas kernel here")


# speedup vs baseline: 138.4766x; 138.4766x over previous
import functools

import jax
import jax.numpy as jnp
from jax import lax
from jax.experimental import pallas as pl
from jax.experimental.pallas import tpu as pltpu
from jax.experimental.pallas import tpu_sc as plsc

B, C, H, W = 4, 96, 224, 224
PLANE = H * W
NPLANES = B * C
NC, NS = 2, 16
NWORKERS = NC * NS
PLANES_PER_W = NPLANES // NWORKERS
L = 16

CHUNK_ROWS = 56
CHUNK = CHUNK_ROWS * W
NCHUNKS = H // CHUNK_ROWS
VECS_PER_ROW = W // L


def _advect_body(img_hbm, u_hbm, v_hbm, out_hbm, img_v, u_v, v_v, acc):
    wid = lax.axis_index("s") * NC + lax.axis_index("c")

    lane = lax.iota(jnp.int32, L).astype(jnp.float32)
    zeros = jnp.zeros((L,), jnp.float32)
    wmax = jnp.float32(W - 1)
    hmax = jnp.float32(H - 1)

    def per_plane(p, _):
        plane = wid * PLANES_PER_W + p
        base_el = plane * PLANE

        def zero_body(i, _):
            acc[pl.ds(i * L, L)] = zeros
            return _
        lax.fori_loop(0, PLANE // L, zero_body, 0, unroll=4)

        def per_chunk(ch, _):
            off = base_el + ch * CHUNK
            pltpu.sync_copy(img_hbm.at[pl.ds(off, CHUNK)], img_v)
            pltpu.sync_copy(u_hbm.at[pl.ds(off, CHUNK)], u_v)
            pltpu.sync_copy(v_hbm.at[pl.ds(off, CHUNK)], v_v)

            def per_row(r, _):
                ybase = (ch * CHUNK_ROWS + r).astype(jnp.float32)
                for j in range(VECS_PER_ROW):
                    sl = pl.ds(r * W + j * L, L)
                    img = img_v[sl]
                    X = jnp.clip(lane + jnp.float32(j * L) + u_v[sl], 0.0, wmax)
                    Y = jnp.clip(ybase + v_v[sl], 0.0, hmax)
                    x0 = X.astype(jnp.int32)
                    y0 = Y.astype(jnp.int32)
                    x1 = jnp.minimum(x0 + 1, W - 1)
                    y1 = jnp.minimum(y0 + 1, H - 1)
                    dx = X - x0.astype(jnp.float32)
                    dy = Y - y0.astype(jnp.float32)
                    gx = x1.astype(jnp.float32) - X
                    gy = y1.astype(jnp.float32) - Y
                    r0 = y0 * W
                    r1 = y1 * W
                    vdx = img * dx
                    vgx = img * gx
                    plsc.addupdate_scatter(acc, [r0 + x0], vdx * dy)
                    plsc.addupdate_scatter(acc, [r1 + x0], vdx * gy)
                    plsc.addupdate_scatter(acc, [r0 + x1], vgx * dy)
                    plsc.addupdate_scatter(acc, [r1 + x1], vgx * gy)
                return _
            lax.fori_loop(0, CHUNK_ROWS, per_row, 0)
            return _
        lax.fori_loop(0, NCHUNKS, per_chunk, 0)

        pltpu.sync_copy(acc, out_hbm.at[pl.ds(base_el, PLANE)])
        return _

    lax.fori_loop(0, PLANES_PER_W, per_plane, 0)


@jax.jit
def kernel(input_image, U, V):
    mesh = plsc.VectorSubcoreMesh(core_axis_name="c", subcore_axis_name="s",
                                  num_cores=NC, num_subcores=NS)
    run = pl.kernel(
        _advect_body,
        out_type=jax.ShapeDtypeStruct((NPLANES * PLANE,), jnp.float32),
        mesh=mesh,
        scratch_types=[
            pltpu.VMEM((CHUNK,), jnp.float32),
            pltpu.VMEM((CHUNK,), jnp.float32),
            pltpu.VMEM((CHUNK,), jnp.float32),
            pltpu.VMEM((PLANE,), jnp.float32),
        ],
        compiler_params=pltpu.CompilerParams(needs_layout_passes=False),
    )
    out = run(input_image.reshape(-1), U.reshape(-1), V.reshape(-1))
    return out.reshape(B, C, H, W)
